# baseline (speedup 1.0000x reference)
import jax
import jax.numpy as jnp
from jax import lax
from jax.experimental import pallas as pl
from jax.experimental.pallas import tpu as pltpu
from jax.experimental.pallas import tpu_sc as plsc

BS, S, H = 16, 2048, 1024
N_PARA, N_SENT = 8, 64
N_OUT = 5
L = 16
HC = H // L
NQ = N_SENT // L
SPW = N_SENT // 2
CHUNK = 4


def _sc_kernel_body(table, packed, wmat, out,
                    pk_v, w_v, wmat_v, rows_v, out_v, sem, sem2, wsem):
    c = lax.axis_index("c")
    s_id = lax.axis_index("s")
    b = s_id
    half = c

    wcp = pltpu.async_copy(wmat, wmat_v, wsem)
    pltpu.sync_copy(packed.at[b], pk_v)

    lane = lax.iota(jnp.int32, L)
    zero_f = jnp.zeros((L,), jnp.float32)
    zero_i = jnp.zeros((L,), jnp.int32)
    one_f = jnp.ones((L,), jnp.float32)

    hbase = half * SPW
    gcp0 = pltpu.async_copy(table.at[pk_v.at[pl.ds(hbase, SPW // 2)]],
                            rows_v.at[pl.ds(0, SPW // 2)], sem)
    gcp1 = pltpu.async_copy(table.at[pk_v.at[pl.ds(hbase + SPW // 2, SPW // 2)]],
                            rows_v.at[pl.ds(SPW // 2, SPW // 2)], sem2)

    off = jnp.full((L,), b * S, jnp.int32)
    sq = [pk_v[pl.ds(q * L, L)] - off for q in range(NQ)]
    w_q = [zero_f for _ in range(NQ)]
    n_valid = zero_f
    for p in range(N_PARA):
        hp = plsc.load_gather(pk_v, [jnp.full((L,), N_SENT + p, jnp.int32)])
        tp = plsc.load_gather(pk_v, [jnp.full((L,), N_SENT + N_PARA + p, jnp.int32)])
        ok = (tp - hp) > 2
        cnt = zero_i
        vcnt = zero_i
        ms = []
        for q in range(NQ):
            m = ok & (hp <= sq[q]) & (sq[q] <= tp)
            ms.append(m)
            cnt = cnt + plsc.all_reduce_population_count(m)
            vcnt = vcnt + plsc.all_reduce_population_count(m & (sq[q] != 0))
        valid_v = jnp.where(vcnt > 0, one_f, zero_f)
        contrib = valid_v / jnp.maximum(cnt.astype(jnp.float32), one_f)
        n_valid = n_valid + valid_v
        for q in range(NQ):
            w_q[q] = w_q[q] + ms[q].astype(jnp.float32) * contrib
    hv_v = jnp.where(n_valid > 0, one_f, zero_f)
    svec = hv_v / jnp.maximum(n_valid, one_f)
    base = (one_f - hv_v) * (1.0 / N_SENT)
    for q in range(NQ):
        w_v[pl.ds(q * L, L)] = w_q[q] * svec + base

    hoff = jnp.full((L,), hbase, jnp.int32)

    def make_hg_body(lo, hi):
        def hg_body(g, outs):
            def s_body(j, accs):
                wb = plsc.load_gather(w_v, [hoff + j])
                return tuple(
                    a + wb * rows_v[j, pl.ds((g * CHUNK + k) * L, L)]
                    for k, a in enumerate(accs)
                )
            accs = lax.fori_loop(lo, hi, s_body, (zero_f,) * CHUNK)
            for k in range(CHUNK):
                outs = tuple(
                    o + accs[k] * wmat_v[i, pl.ds((g * CHUNK + k) * L, L)]
                    for i, o in enumerate(outs)
                )
            return outs
        return hg_body

    gcp0.wait()
    wcp.wait()
    outs = lax.fori_loop(0, HC // CHUNK, make_hg_body(0, SPW // 2),
                         (zero_f,) * N_OUT)
    gcp1.wait()
    outs = lax.fori_loop(0, HC // CHUNK, make_hg_body(SPW // 2, SPW),
                         outs)

    res = zero_f
    for o in range(N_OUT):
        dvec = jnp.full((L,), jnp.sum(outs[o], axis=0), jnp.float32)
        res = res + jnp.where(lane == o, dvec, zero_f)
    out_v[...] = res
    pltpu.sync_copy(out_v, out.at[half * BS + b])


@jax.jit
def kernel(backbone_outputs, attention_mask, paragraph_head_idxs, paragraph_tail_idxs,
           paragraph_attention_mask, sentence_head_idxs, sentence_tail_idxs,
           sentence_attention_mask, W, b):
    del attention_mask, paragraph_attention_mask, sentence_tail_idxs, sentence_attention_mask
    table = backbone_outputs.reshape(BS * S, H)
    row_base = (jnp.arange(BS, dtype=jnp.int32) * S)[:, None]
    packed = jnp.concatenate(
        [sentence_head_idxs.astype(jnp.int32) + row_base,
         paragraph_head_idxs.astype(jnp.int32),
         paragraph_tail_idxs.astype(jnp.int32)], axis=1)

    mesh = plsc.VectorSubcoreMesh(core_axis_name="c", subcore_axis_name="s")
    out_pad = pl.kernel(
        _sc_kernel_body,
        mesh=mesh,
        compiler_params=pltpu.CompilerParams(
            needs_layout_passes=False,
            disable_bounds_checks=True,
            disable_semaphore_checks=True,
            skip_device_barrier=True,
        ),
        out_type=jax.ShapeDtypeStruct((2 * BS, L), jnp.float32),
        scratch_types=[
            pltpu.VMEM((N_SENT + L,), jnp.int32),
            pltpu.VMEM((N_SENT,), jnp.float32),
            pltpu.VMEM((N_OUT, H), jnp.float32),
            pltpu.VMEM((SPW, H), jnp.float32),
            pltpu.VMEM((L,), jnp.float32),
            pltpu.SemaphoreType.DMA,
            pltpu.SemaphoreType.DMA,
            pltpu.SemaphoreType.DMA,
        ],
    )(table, packed, W.astype(jnp.float32))
    halves = out_pad[:BS, :N_OUT] + out_pad[BS:, :N_OUT]
    return halves + b.astype(jnp.float32)[None, :]


# speedup vs baseline: 1.0054x; 1.0054x over previous
import jax
import jax.numpy as jnp
from jax import lax
from jax.experimental import pallas as pl
from jax.experimental.pallas import tpu as pltpu
from jax.experimental.pallas import tpu_sc as plsc

BS, S, H = 16, 2048, 1024
N_PARA, N_SENT = 8, 64
N_OUT = 5
L = 16
HC = H // L
NQ = N_SENT // L
SPW = N_SENT // 2
CHUNK = 4


def _sc_kernel_body(table, packed, wmat, out,
                    pk_v, w_v, wmat_v, rows_v, out_v, sem, sem2, wsem):
    c = lax.axis_index("c")
    s_id = lax.axis_index("s")
    b = s_id
    half = c

    wcp = pltpu.async_copy(wmat, wmat_v, wsem)
    pltpu.sync_copy(packed.at[b], pk_v)

    lane = lax.iota(jnp.int32, L)
    zero_f = jnp.zeros((L,), jnp.float32)
    zero_i = jnp.zeros((L,), jnp.int32)
    one_f = jnp.ones((L,), jnp.float32)

    hbase = half * SPW
    gcp0 = pltpu.async_copy(table.at[pk_v.at[pl.ds(hbase, SPW // 2)]],
                            rows_v.at[pl.ds(0, SPW // 2)], sem)
    gcp1 = pltpu.async_copy(table.at[pk_v.at[pl.ds(hbase + SPW // 2, SPW // 2)]],
                            rows_v.at[pl.ds(SPW // 2, SPW // 2)], sem2)

    off = jnp.full((L,), b * S, jnp.int32)
    sq = [pk_v[pl.ds(q * L, L)] - off for q in range(NQ)]
    w_q = [zero_f for _ in range(NQ)]
    n_valid = zero_f
    for p in range(N_PARA):
        hp = plsc.load_gather(pk_v, [jnp.full((L,), N_SENT + p, jnp.int32)])
        tp = plsc.load_gather(pk_v, [jnp.full((L,), N_SENT + N_PARA + p, jnp.int32)])
        ok = (tp - hp) > 2
        cnt = zero_i
        vcnt = zero_i
        ms = []
        for q in range(NQ):
            m = ok & (hp <= sq[q]) & (sq[q] <= tp)
            ms.append(m)
            cnt = cnt + plsc.all_reduce_population_count(m)
            vcnt = vcnt + plsc.all_reduce_population_count(m & (sq[q] != 0))
        valid_v = jnp.where(vcnt > 0, one_f, zero_f)
        contrib = valid_v / jnp.maximum(cnt.astype(jnp.float32), one_f)
        n_valid = n_valid + valid_v
        for q in range(NQ):
            w_q[q] = w_q[q] + ms[q].astype(jnp.float32) * contrib
    hv_v = jnp.where(n_valid > 0, one_f, zero_f)
    svec = hv_v / jnp.maximum(n_valid, one_f)
    base = (one_f - hv_v) * (1.0 / N_SENT)
    for q in range(NQ):
        w_v[pl.ds(q * L, L)] = w_q[q] * svec + base

    hoff = jnp.full((L,), hbase, jnp.int32)

    def make_hg_body(lo, hi):
        def hg_body(g, outs):
            def s_body(j, accs):
                wb = plsc.load_gather(w_v, [hoff + j])
                return tuple(
                    a + wb * rows_v[j, pl.ds((g * CHUNK + k) * L, L)]
                    for k, a in enumerate(accs)
                )
            accs = lax.fori_loop(lo, hi, s_body, (zero_f,) * CHUNK, unroll=4)
            for k in range(CHUNK):
                outs = tuple(
                    o + accs[k] * wmat_v[i, pl.ds((g * CHUNK + k) * L, L)]
                    for i, o in enumerate(outs)
                )
            return outs
        return hg_body

    gcp0.wait()
    wcp.wait()
    outs = lax.fori_loop(0, HC // CHUNK, make_hg_body(0, SPW // 2),
                         (zero_f,) * N_OUT)
    gcp1.wait()
    outs = lax.fori_loop(0, HC // CHUNK, make_hg_body(SPW // 2, SPW),
                         outs)

    res = zero_f
    for o in range(N_OUT):
        dvec = jnp.full((L,), jnp.sum(outs[o], axis=0), jnp.float32)
        res = res + jnp.where(lane == o, dvec, zero_f)
    out_v[...] = res
    pltpu.sync_copy(out_v, out.at[half * BS + b])


@jax.jit
def kernel(backbone_outputs, attention_mask, paragraph_head_idxs, paragraph_tail_idxs,
           paragraph_attention_mask, sentence_head_idxs, sentence_tail_idxs,
           sentence_attention_mask, W, b):
    del attention_mask, paragraph_attention_mask, sentence_tail_idxs, sentence_attention_mask
    table = backbone_outputs.reshape(BS * S, H)
    row_base = (jnp.arange(BS, dtype=jnp.int32) * S)[:, None]
    packed = jnp.concatenate(
        [sentence_head_idxs.astype(jnp.int32) + row_base,
         paragraph_head_idxs.astype(jnp.int32),
         paragraph_tail_idxs.astype(jnp.int32)], axis=1)

    mesh = plsc.VectorSubcoreMesh(core_axis_name="c", subcore_axis_name="s")
    out_pad = pl.kernel(
        _sc_kernel_body,
        mesh=mesh,
        compiler_params=pltpu.CompilerParams(
            needs_layout_passes=False,
            disable_bounds_checks=True,
            disable_semaphore_checks=True,
            skip_device_barrier=True,
        ),
        out_type=jax.ShapeDtypeStruct((2 * BS, L), jnp.float32),
        scratch_types=[
            pltpu.VMEM((N_SENT + L,), jnp.int32),
            pltpu.VMEM((N_SENT,), jnp.float32),
            pltpu.VMEM((N_OUT, H), jnp.float32),
            pltpu.VMEM((SPW, H), jnp.float32),
            pltpu.VMEM((L,), jnp.float32),
            pltpu.SemaphoreType.DMA,
            pltpu.SemaphoreType.DMA,
            pltpu.SemaphoreType.DMA,
        ],
    )(table, packed, W.astype(jnp.float32))
    halves = out_pad[:BS, :N_OUT] + out_pad[BS:, :N_OUT]
    return halves + b.astype(jnp.float32)[None, :]


# speedup vs baseline: 1.0059x; 1.0005x over previous
import jax
import jax.numpy as jnp
from jax import lax
from jax.experimental import pallas as pl
from jax.experimental.pallas import tpu as pltpu
from jax.experimental.pallas import tpu_sc as plsc

BS, S, H = 16, 2048, 1024
N_PARA, N_SENT = 8, 64
N_OUT = 5
L = 16
HC = H // L
NQ = N_SENT // L
SPW = N_SENT // 2
CHUNK = 4


def _sc_kernel_body(table, sh, ph, pt, wmat, out,
                    pk_v, gidx_v, ph_v, pt_v, w_v, wmat_v, rows_v, out_v,
                    sem, sem2, wsem):
    c = lax.axis_index("c")
    s_id = lax.axis_index("s")
    b = s_id
    half = c

    wcp = pltpu.async_copy(wmat, wmat_v, wsem)
    pltpu.sync_copy(sh.at[b], pk_v)

    lane = lax.iota(jnp.int32, L)
    zero_f = jnp.zeros((L,), jnp.float32)
    zero_i = jnp.zeros((L,), jnp.int32)
    one_f = jnp.ones((L,), jnp.float32)

    off = jnp.full((L,), b * S, jnp.int32)
    hbase = half * SPW
    for j in range(SPW // L):
        gidx_v[pl.ds(j * L, L)] = pk_v[pl.ds(hbase + j * L, L)] + off
    gcp0 = pltpu.async_copy(table.at[gidx_v.at[pl.ds(0, SPW // 2)]],
                            rows_v.at[pl.ds(0, SPW // 2)], sem)
    gcp1 = pltpu.async_copy(table.at[gidx_v.at[pl.ds(SPW // 2, SPW // 2)]],
                            rows_v.at[pl.ds(SPW // 2, SPW // 2)], sem2)

    pltpu.sync_copy(ph.at[b], ph_v)
    pltpu.sync_copy(pt.at[b], pt_v)

    sq = [pk_v[pl.ds(q * L, L)] for q in range(NQ)]
    w_q = [zero_f for _ in range(NQ)]
    n_valid = zero_f
    for p in range(N_PARA):
        hp = plsc.load_gather(ph_v, [jnp.full((L,), p, jnp.int32)])
        tp = plsc.load_gather(pt_v, [jnp.full((L,), p, jnp.int32)])
        ok = (tp - hp) > 2
        cnt = zero_i
        vcnt = zero_i
        ms = []
        for q in range(NQ):
            m = ok & (hp <= sq[q]) & (sq[q] <= tp)
            ms.append(m)
            cnt = cnt + plsc.all_reduce_population_count(m)
            vcnt = vcnt + plsc.all_reduce_population_count(m & (sq[q] != 0))
        valid_v = jnp.where(vcnt > 0, one_f, zero_f)
        contrib = valid_v / jnp.maximum(cnt.astype(jnp.float32), one_f)
        n_valid = n_valid + valid_v
        for q in range(NQ):
            w_q[q] = w_q[q] + ms[q].astype(jnp.float32) * contrib
    hv_v = jnp.where(n_valid > 0, one_f, zero_f)
    svec = hv_v / jnp.maximum(n_valid, one_f)
    base = (one_f - hv_v) * (1.0 / N_SENT)
    for q in range(NQ):
        w_v[pl.ds(q * L, L)] = w_q[q] * svec + base

    hoff = jnp.full((L,), hbase, jnp.int32)

    def make_hg_body(lo, hi):
        def hg_body(g, outs):
            def s_body(j, accs):
                wb = plsc.load_gather(w_v, [hoff + j])
                return tuple(
                    a + wb * rows_v[j, pl.ds((g * CHUNK + k) * L, L)]
                    for k, a in enumerate(accs)
                )
            accs = lax.fori_loop(lo, hi, s_body, (zero_f,) * CHUNK, unroll=4)
            for k in range(CHUNK):
                outs = tuple(
                    o + accs[k] * wmat_v[i, pl.ds((g * CHUNK + k) * L, L)]
                    for i, o in enumerate(outs)
                )
            return outs
        return hg_body

    gcp0.wait()
    wcp.wait()
    outs = lax.fori_loop(0, HC // CHUNK, make_hg_body(0, SPW // 2),
                         (zero_f,) * N_OUT)
    gcp1.wait()
    outs = lax.fori_loop(0, HC // CHUNK, make_hg_body(SPW // 2, SPW),
                         outs)

    res = zero_f
    for o in range(N_OUT):
        dvec = jnp.full((L,), jnp.sum(outs[o], axis=0), jnp.float32)
        res = res + jnp.where(lane == o, dvec, zero_f)
    out_v[...] = res
    pltpu.sync_copy(out_v, out.at[half * BS + b])


@jax.jit
def kernel(backbone_outputs, attention_mask, paragraph_head_idxs, paragraph_tail_idxs,
           paragraph_attention_mask, sentence_head_idxs, sentence_tail_idxs,
           sentence_attention_mask, W, b):
    del attention_mask, paragraph_attention_mask, sentence_tail_idxs, sentence_attention_mask
    table = backbone_outputs.reshape(BS * S, H)

    mesh = plsc.VectorSubcoreMesh(core_axis_name="c", subcore_axis_name="s")
    out_pad = pl.kernel(
        _sc_kernel_body,
        mesh=mesh,
        compiler_params=pltpu.CompilerParams(
            needs_layout_passes=False,
            disable_bounds_checks=True,
            disable_semaphore_checks=True,
            skip_device_barrier=True,
        ),
        out_type=jax.ShapeDtypeStruct((2 * BS, L), jnp.float32),
        scratch_types=[
            pltpu.VMEM((N_SENT,), jnp.int32),
            pltpu.VMEM((SPW,), jnp.int32),
            pltpu.VMEM((N_PARA,), jnp.int32),
            pltpu.VMEM((N_PARA,), jnp.int32),
            pltpu.VMEM((N_SENT,), jnp.float32),
            pltpu.VMEM((N_OUT, H), jnp.float32),
            pltpu.VMEM((SPW, H), jnp.float32),
            pltpu.VMEM((L,), jnp.float32),
            pltpu.SemaphoreType.DMA,
            pltpu.SemaphoreType.DMA,
            pltpu.SemaphoreType.DMA,
        ],
    )(table, sentence_head_idxs, paragraph_head_idxs, paragraph_tail_idxs,
      W.astype(jnp.float32))
    halves = out_pad[:BS, :N_OUT] + out_pad[BS:, :N_OUT]
    return halves + b.astype(jnp.float32)[None, :]


# speedup vs baseline: 1.0147x; 1.0088x over previous
import jax
import jax.numpy as jnp
from jax import lax
from jax.experimental import pallas as pl
from jax.experimental.pallas import tpu as pltpu
from jax.experimental.pallas import tpu_sc as plsc

BS, S, H = 16, 2048, 1024
N_PARA, N_SENT = 8, 64
N_OUT = 5
L = 16
HC = H // L
NQ = N_SENT // L
SPW = N_SENT // 2
CHUNK = 8


def _sc_kernel_body(table, packed, wmat, out,
                    pk_v, w_v, wmat_v, rows_v, out_v, sem, sem2, wsem):
    c = lax.axis_index("c")
    s_id = lax.axis_index("s")
    b = s_id
    half = c

    wcp = pltpu.async_copy(wmat, wmat_v, wsem)
    pltpu.sync_copy(packed.at[b], pk_v)

    lane = lax.iota(jnp.int32, L)
    zero_f = jnp.zeros((L,), jnp.float32)
    zero_i = jnp.zeros((L,), jnp.int32)
    one_f = jnp.ones((L,), jnp.float32)

    hbase = half * SPW
    gcp0 = pltpu.async_copy(table.at[pk_v.at[pl.ds(hbase, SPW // 2)]],
                            rows_v.at[pl.ds(0, SPW // 2)], sem)
    gcp1 = pltpu.async_copy(table.at[pk_v.at[pl.ds(hbase + SPW // 2, SPW // 2)]],
                            rows_v.at[pl.ds(SPW // 2, SPW // 2)], sem2)

    off = jnp.full((L,), b * S, jnp.int32)
    sq = [pk_v[pl.ds(q * L, L)] - off for q in range(NQ)]
    w_q = [zero_f for _ in range(NQ)]
    n_valid = zero_f
    for p in range(N_PARA):
        hp = plsc.load_gather(pk_v, [jnp.full((L,), N_SENT + p, jnp.int32)])
        tp = plsc.load_gather(pk_v, [jnp.full((L,), N_SENT + N_PARA + p, jnp.int32)])
        ok = (tp - hp) > 2
        cnt = zero_i
        vcnt = zero_i
        ms = []
        for q in range(NQ):
            m = ok & (hp <= sq[q]) & (sq[q] <= tp)
            ms.append(m)
            cnt = cnt + plsc.all_reduce_population_count(m)
            vcnt = vcnt + plsc.all_reduce_population_count(m & (sq[q] != 0))
        valid_v = jnp.where(vcnt > 0, one_f, zero_f)
        contrib = valid_v / jnp.maximum(cnt.astype(jnp.float32), one_f)
        n_valid = n_valid + valid_v
        for q in range(NQ):
            w_q[q] = w_q[q] + ms[q].astype(jnp.float32) * contrib
    hv_v = jnp.where(n_valid > 0, one_f, zero_f)
    svec = hv_v / jnp.maximum(n_valid, one_f)
    base = (one_f - hv_v) * (1.0 / N_SENT)
    for q in range(NQ):
        w_v[pl.ds(q * L, L)] = w_q[q] * svec + base

    hoff = jnp.full((L,), hbase, jnp.int32)

    def make_hg_body(lo, hi):
        def hg_body(g, outs):
            def s_body(j, accs):
                wb = plsc.load_gather(w_v, [hoff + j])
                return tuple(
                    a + wb * rows_v[j, pl.ds((g * CHUNK + k) * L, L)]
                    for k, a in enumerate(accs)
                )
            accs = lax.fori_loop(lo, hi, s_body, (zero_f,) * CHUNK, unroll=2)
            for k in range(CHUNK):
                outs = tuple(
                    o + accs[k] * wmat_v[i, pl.ds((g * CHUNK + k) * L, L)]
                    for i, o in enumerate(outs)
                )
            return outs
        return hg_body

    gcp0.wait()
    wcp.wait()
    outs = lax.fori_loop(0, HC // CHUNK, make_hg_body(0, SPW // 2),
                         (zero_f,) * N_OUT)
    gcp1.wait()
    outs = lax.fori_loop(0, HC // CHUNK, make_hg_body(SPW // 2, SPW),
                         outs)

    res = zero_f
    for o in range(N_OUT):
        dvec = jnp.full((L,), jnp.sum(outs[o], axis=0), jnp.float32)
        res = res + jnp.where(lane == o, dvec, zero_f)
    out_v[...] = res
    pltpu.sync_copy(out_v, out.at[half * BS + b])


@jax.jit
def kernel(backbone_outputs, attention_mask, paragraph_head_idxs, paragraph_tail_idxs,
           paragraph_attention_mask, sentence_head_idxs, sentence_tail_idxs,
           sentence_attention_mask, W, b):
    del attention_mask, paragraph_attention_mask, sentence_tail_idxs, sentence_attention_mask
    table = backbone_outputs.reshape(BS * S, H)
    row_base = (jnp.arange(BS, dtype=jnp.int32) * S)[:, None]
    packed = jnp.concatenate(
        [sentence_head_idxs.astype(jnp.int32) + row_base,
         paragraph_head_idxs.astype(jnp.int32),
         paragraph_tail_idxs.astype(jnp.int32)], axis=1)

    mesh = plsc.VectorSubcoreMesh(core_axis_name="c", subcore_axis_name="s")
    out_pad = pl.kernel(
        _sc_kernel_body,
        mesh=mesh,
        compiler_params=pltpu.CompilerParams(
            needs_layout_passes=False,
            disable_bounds_checks=True,
            disable_semaphore_checks=True,
            skip_device_barrier=True,
        ),
        out_type=jax.ShapeDtypeStruct((2 * BS, L), jnp.float32),
        scratch_types=[
            pltpu.VMEM((N_SENT + L,), jnp.int32),
            pltpu.VMEM((N_SENT,), jnp.float32),
            pltpu.VMEM((N_OUT, H), jnp.float32),
            pltpu.VMEM((SPW, H), jnp.float32),
            pltpu.VMEM((L,), jnp.float32),
            pltpu.SemaphoreType.DMA,
            pltpu.SemaphoreType.DMA,
            pltpu.SemaphoreType.DMA,
        ],
    )(table, packed, W.astype(jnp.float32))
    halves = out_pad[:BS, :N_OUT] + out_pad[BS:, :N_OUT]
    return halves + b.astype(jnp.float32)[None, :]
